# Initial kernel scaffold; baseline (speedup 1.0000x reference)
import jax
import jax.numpy as jnp
from jax.experimental import pallas as pl


def kernel(a, p, n, emb_table):
    raise NotImplementedError("write your pallas kernel here")


# speedup vs baseline: 4.8729x; 4.8729x over previous
import functools

import jax
import jax.numpy as jnp
from jax import lax
from jax.experimental import pallas as pl
from jax.experimental.pallas import tpu as pltpu
from jax.experimental.pallas import tpu_sc as plsc

_NUM_EMB = 1000
_EMB_DIM = 128
_BATCH = 16384
_MARGIN = 1.0
_EPS = 1e-6
_NPAD = 1024

_NC = 2
_NS = 16
_NW = _NC * _NS
_BPW = _BATCH // _NW
_CH = 128
_NCH = _BPW // _CH
_LANES = 16


def _dist_matrix_body(t_ref, d_ref):
    t = t_ref[...]
    sq = t * t
    n2c = jnp.sum(sq, axis=1, keepdims=True)
    rsc = jnp.sum(t, axis=1, keepdims=True)
    ones = jnp.ones((1, _EMB_DIM), jnp.float32)
    dn = (((1,), (1,)), ((), ()))
    hi = jax.lax.Precision.HIGHEST
    n2r = lax.dot_general(ones, sq, dn, precision=hi)
    rsr = lax.dot_general(ones, t, dn, precision=hi)
    g = lax.dot_general(t, t, dn, precision=hi)
    diag_val = _EMB_DIM * _EPS * _EPS
    d2 = n2c + n2r - 2.0 * g + (2.0 * _EPS) * (rsc - rsr) + diag_val
    row = lax.broadcasted_iota(jnp.int32, (_NPAD, _NPAD), 0)
    col = lax.broadcasted_iota(jnp.int32, (_NPAD, _NPAD), 1)
    d2 = jnp.where(row == col, diag_val, d2)
    d_ref[...] = jnp.sqrt(jnp.maximum(d2, 0.0))


def _sc_triplet_body(dflat, a_hbm, p_hbm, n_hbm, out_hbm,
                     a_v, p_v, n_v, iap, ian, dap, dan, accv, sem1, sem2):
    wid = lax.axis_index("s") * _NC + lax.axis_index("c")
    base = wid * _BPW
    pltpu.sync_copy(a_hbm.at[pl.ds(base, _BPW)], a_v)
    pltpu.sync_copy(p_hbm.at[pl.ds(base, _BPW)], p_v)
    pltpu.sync_copy(n_hbm.at[pl.ds(base, _BPW)], n_v)
    for v in range(_BPW // _LANES):
        s = v * _LANES
        arow = a_v[pl.ds(s, _LANES)] * _NPAD
        iap[pl.ds(s, _LANES)] = arow + p_v[pl.ds(s, _LANES)]
        ian[pl.ds(s, _LANES)] = arow + n_v[pl.ds(s, _LANES)]
    copies = []
    for c in range(_NCH):
        sl = pl.ds(c * _CH, _CH)
        copies.append(pltpu.async_copy(dflat.at[iap.at[sl]], dap.at[sl], sem1))
        copies.append(pltpu.async_copy(dflat.at[ian.at[sl]], dan.at[sl], sem2))
    for cp in copies:
        cp.wait()
    acc = jnp.zeros((_LANES,), jnp.float32)
    for v in range(_BPW // _LANES):
        sl = pl.ds(v * _LANES, _LANES)
        acc = acc + jnp.maximum(dap[sl] - dan[sl] + _MARGIN, 0.0)
    accv[...] = acc
    pltpu.sync_copy(accv, out_hbm.at[wid])


_sc_triplet = functools.partial(
    pl.kernel,
    out_type=jax.ShapeDtypeStruct((_NW, _LANES), jnp.float32),
    mesh=plsc.VectorSubcoreMesh(core_axis_name="c", subcore_axis_name="s"),
    scratch_types=[
        pltpu.VMEM((_BPW,), jnp.int32),
        pltpu.VMEM((_BPW,), jnp.int32),
        pltpu.VMEM((_BPW,), jnp.int32),
        pltpu.VMEM((_BPW,), jnp.int32),
        pltpu.VMEM((_BPW,), jnp.int32),
        pltpu.VMEM((_BPW,), jnp.float32),
        pltpu.VMEM((_BPW,), jnp.float32),
        pltpu.VMEM((_LANES,), jnp.float32),
        pltpu.SemaphoreType.DMA,
        pltpu.SemaphoreType.DMA,
    ],
)(_sc_triplet_body)


def kernel(a, p, n, emb_table):
    t_pad = jnp.zeros((_NPAD, _EMB_DIM), jnp.float32).at[:_NUM_EMB].set(emb_table)
    dist = pl.pallas_call(
        _dist_matrix_body,
        out_shape=jax.ShapeDtypeStruct((_NPAD, _NPAD), jnp.float32),
    )(t_pad)
    partials = _sc_triplet(dist.reshape(_NPAD * _NPAD), a, p, n)
    return jnp.sum(partials) / _BATCH


# speedup vs baseline: 6.5315x; 1.3404x over previous
import functools

import jax
import jax.numpy as jnp
from jax import lax
from jax.experimental import pallas as pl
from jax.experimental.pallas import tpu as pltpu
from jax.experimental.pallas import tpu_sc as plsc

_NUM_EMB = 1000
_EMB_DIM = 128
_BATCH = 16384
_MARGIN = 1.0
_EPS = 1e-6
_NPAD = 1024

_NC = 2
_NS = 16
_NW = _NC * _NS
_BPW = _BATCH // _NW
_CH = 128
_NCH = _BPW // _CH
_LANES = 16


def _dist_matrix_body(t_ref, d_ref):
    t = jnp.concatenate(
        [t_ref[...], jnp.zeros((_NPAD - _NUM_EMB, _EMB_DIM), jnp.float32)], axis=0)
    sq = t * t
    n2c = jnp.sum(sq, axis=1, keepdims=True)
    rsc = jnp.sum(t, axis=1, keepdims=True)
    ones = jnp.ones((1, _EMB_DIM), jnp.float32)
    dn = (((1,), (1,)), ((), ()))
    n2r = lax.dot_general(ones, sq, dn)
    rsr = lax.dot_general(ones, t, dn)
    g = lax.dot_general(t, t, dn)
    diag_val = _EMB_DIM * _EPS * _EPS
    d2 = n2c + n2r - 2.0 * g + (2.0 * _EPS) * (rsc - rsr) + diag_val
    row = lax.broadcasted_iota(jnp.int32, (_NPAD, _NPAD), 0)
    col = lax.broadcasted_iota(jnp.int32, (_NPAD, _NPAD), 1)
    d2 = jnp.where(row == col, diag_val, d2)
    d = jnp.sqrt(jnp.maximum(d2, 0.0))
    d_ref[...] = d.reshape(_NPAD * 8, 128)


def _sc_triplet_body(dflat, a_hbm, p_hbm, n_hbm, out_hbm,
                     a_v, p_v, n_v, iap, ian, dap, dan, accv, sem1, sem2):
    wid = lax.axis_index("s") * _NC + lax.axis_index("c")
    base = wid * _BPW
    pltpu.sync_copy(a_hbm.at[pl.ds(base, _BPW)], a_v)
    pltpu.sync_copy(p_hbm.at[pl.ds(base, _BPW)], p_v)
    pltpu.sync_copy(n_hbm.at[pl.ds(base, _BPW)], n_v)
    for v in range(_BPW // _LANES):
        s = v * _LANES
        arow = a_v[pl.ds(s, _LANES)] * _NPAD
        iap[pl.ds(s, _LANES)] = arow + p_v[pl.ds(s, _LANES)]
        ian[pl.ds(s, _LANES)] = arow + n_v[pl.ds(s, _LANES)]
    copies = []
    for c in range(_NCH):
        sl = pl.ds(c * _CH, _CH)
        copies.append(pltpu.async_copy(dflat.at[iap.at[sl]], dap.at[sl], sem1))
        copies.append(pltpu.async_copy(dflat.at[ian.at[sl]], dan.at[sl], sem2))
    for cp in copies:
        cp.wait()
    acc = jnp.zeros((_LANES,), jnp.float32)
    for v in range(_BPW // _LANES):
        sl = pl.ds(v * _LANES, _LANES)
        acc = acc + jnp.maximum(dap[sl] - dan[sl] + _MARGIN, 0.0)
    accv[...] = acc
    pltpu.sync_copy(accv, out_hbm.at[wid])


_sc_triplet = functools.partial(
    pl.kernel,
    out_type=jax.ShapeDtypeStruct((_NW, _LANES), jnp.float32),
    mesh=plsc.VectorSubcoreMesh(core_axis_name="c", subcore_axis_name="s"),
    scratch_types=[
        pltpu.VMEM((_BPW,), jnp.int32),
        pltpu.VMEM((_BPW,), jnp.int32),
        pltpu.VMEM((_BPW,), jnp.int32),
        pltpu.VMEM((_BPW,), jnp.int32),
        pltpu.VMEM((_BPW,), jnp.int32),
        pltpu.VMEM((_BPW,), jnp.float32),
        pltpu.VMEM((_BPW,), jnp.float32),
        pltpu.VMEM((_LANES,), jnp.float32),
        pltpu.SemaphoreType.DMA,
        pltpu.SemaphoreType.DMA,
    ],
)(_sc_triplet_body)


def kernel(a, p, n, emb_table):
    dist = pl.pallas_call(
        _dist_matrix_body,
        out_shape=jax.ShapeDtypeStruct((_NPAD * 8, 128), jnp.float32),
    )(emb_table)
    partials = _sc_triplet(dist.reshape(_NPAD * _NPAD), a, p, n)
    return jnp.sum(partials) / _BATCH


# speedup vs baseline: 6.9130x; 1.0584x over previous
import functools

import jax
import jax.numpy as jnp
from jax import lax
from jax.experimental import pallas as pl
from jax.experimental.pallas import tpu as pltpu
from jax.experimental.pallas import tpu_sc as plsc

_NUM_EMB = 1000
_EMB_DIM = 128
_BATCH = 16384
_MARGIN = 1.0
_EPS = 1e-6
_NPAD = 1024

_NC = 2
_NS = 16
_NW = _NC * _NS
_BPW = _BATCH // _NW
_CH = 128
_NCH = _BPW // _CH
_LANES = 16


def _dist_matrix_body(t_ref, d_ref):
    t = jnp.concatenate(
        [t_ref[...], jnp.zeros((_NPAD - _NUM_EMB, _EMB_DIM), jnp.float32)], axis=0)
    sq = t * t
    n2c = jnp.sum(sq, axis=1, keepdims=True)
    rsc = jnp.sum(t, axis=1, keepdims=True)
    ones = jnp.ones((1, _EMB_DIM), jnp.float32)
    dn = (((1,), (1,)), ((), ()))
    n2r = lax.dot_general(ones, sq, dn)
    rsr = lax.dot_general(ones, t, dn)
    g = lax.dot_general(t, -2.0 * t, dn)
    diag_val = _EMB_DIM * _EPS * _EPS
    u = n2c + (2.0 * _EPS) * rsc
    v = (n2r - (2.0 * _EPS) * rsr) + diag_val
    d2 = (u + g) + v
    d = jnp.sqrt(jnp.maximum(d2, 0.0))
    d_ref[...] = d.reshape(_NPAD * 8, 128)


def _sc_triplet_body(dflat, a_hbm, p_hbm, n_hbm, out_hbm,
                     a_v, p_v, n_v, iap, ian, dap, dan, accv, sem1, sem2):
    wid = lax.axis_index("s") * _NC + lax.axis_index("c")
    base = wid * _BPW
    cpa = pltpu.async_copy(a_hbm.at[pl.ds(base, _BPW)], a_v, sem1)
    cpp = pltpu.async_copy(p_hbm.at[pl.ds(base, _BPW)], p_v, sem2)
    cpn = pltpu.async_copy(n_hbm.at[pl.ds(base, _BPW)], n_v, sem1)
    cpa.wait()
    cpp.wait()
    copies = []
    for c in range(_NCH):
        for v in range(_CH // _LANES):
            s = c * _CH + v * _LANES
            arow = a_v[pl.ds(s, _LANES)] * _NPAD
            iap[pl.ds(s, _LANES)] = arow + p_v[pl.ds(s, _LANES)]
        sl = pl.ds(c * _CH, _CH)
        copies.append(pltpu.async_copy(dflat.at[iap.at[sl]], dap.at[sl], sem2))
    cpn.wait()
    for c in range(_NCH):
        for v in range(_CH // _LANES):
            s = c * _CH + v * _LANES
            ian[pl.ds(s, _LANES)] = a_v[pl.ds(s, _LANES)] * _NPAD + n_v[pl.ds(s, _LANES)]
        sl = pl.ds(c * _CH, _CH)
        copies.append(pltpu.async_copy(dflat.at[ian.at[sl]], dan.at[sl], sem1))
    for cp in copies:
        cp.wait()
    acc = jnp.zeros((_LANES,), jnp.float32)
    for v in range(_BPW // _LANES):
        sl = pl.ds(v * _LANES, _LANES)
        acc = acc + jnp.maximum(dap[sl] - dan[sl] + _MARGIN, 0.0)
    accv[...] = acc
    pltpu.sync_copy(accv, out_hbm.at[wid])


_sc_triplet = functools.partial(
    pl.kernel,
    out_type=jax.ShapeDtypeStruct((_NW, _LANES), jnp.float32),
    mesh=plsc.VectorSubcoreMesh(core_axis_name="c", subcore_axis_name="s"),
    scratch_types=[
        pltpu.VMEM((_BPW,), jnp.int32),
        pltpu.VMEM((_BPW,), jnp.int32),
        pltpu.VMEM((_BPW,), jnp.int32),
        pltpu.VMEM((_BPW,), jnp.int32),
        pltpu.VMEM((_BPW,), jnp.int32),
        pltpu.VMEM((_BPW,), jnp.float32),
        pltpu.VMEM((_BPW,), jnp.float32),
        pltpu.VMEM((_LANES,), jnp.float32),
        pltpu.SemaphoreType.DMA,
        pltpu.SemaphoreType.DMA,
    ],
)(_sc_triplet_body)


def kernel(a, p, n, emb_table):
    dist = pl.pallas_call(
        _dist_matrix_body,
        out_shape=jax.ShapeDtypeStruct((_NPAD * 8, 128), jnp.float32),
    )(emb_table)
    partials = _sc_triplet(dist.reshape(_NPAD * _NPAD), a, p, n)
    return jnp.sum(partials) / _BATCH
